# Initial kernel scaffold; baseline (speedup 1.0000x reference)
import jax
import jax.numpy as jnp
from jax.experimental import pallas as pl


def kernel(inputs, targets, entire_inputs, edge_index, W1, W2, Wf, bf, Wo, Wx, bx, Wh, Wdec):
    raise NotImplementedError("write your pallas kernel here")


# speedup vs baseline: 15.5045x; 15.5045x over previous
import functools

import jax
import jax.numpy as jnp
from jax import lax
from jax.experimental import pallas as pl
from jax.experimental.pallas import tpu as pltpu
from jax.experimental.pallas import tpu_sc as plsc

N = 10000
E = 320000
B = 4
TIN = 12
TOUT = 12
DH = 64

NC = 2
NS = 16
NW = NC * NS
EW = E // NW
K = 80
NCH = EW // K
G = K // 16
DUMMY = N
AGGR = 10240
RPT = AGGR // NS

BLK_A = 256
BLK_C = 256


def _enc_body(ein_ref, w1_ref, w2_ref, wfa_ref, wfb_ref, bf_ref,
              ha_ref, hb_ref):
    h1 = jnp.maximum(
        jnp.dot(ein_ref[...], w1_ref[...],
                preferred_element_type=jnp.float32), 0.0)
    h = jnp.maximum(
        jnp.dot(h1, w2_ref[...], preferred_element_type=jnp.float32), 0.0)
    ha_ref[...] = (
        jnp.dot(h, wfa_ref[...], preferred_element_type=jnp.float32)
        + bf_ref[...])
    hb_ref[...] = jnp.dot(h, wfb_ref[...],
                          preferred_element_type=jnp.float32)


def _encoder(ein, w1, w2, wfa, wfb, bf2):
    t = ein.shape[1]
    grid = pl.cdiv(N, BLK_A)
    return pl.pallas_call(
        _enc_body,
        grid=(grid,),
        in_specs=[
            pl.BlockSpec((BLK_A, t), lambda i: (i, 0)),
            pl.BlockSpec((t, 256), lambda i: (0, 0)),
            pl.BlockSpec((256, 128), lambda i: (0, 0)),
            pl.BlockSpec((128, DH), lambda i: (0, 0)),
            pl.BlockSpec((128, DH), lambda i: (0, 0)),
            pl.BlockSpec((1, DH), lambda i: (0, 0)),
        ],
        out_specs=[
            pl.BlockSpec((BLK_A, DH), lambda i: (i, 0)),
            pl.BlockSpec((BLK_A, DH), lambda i: (i, 0)),
        ],
        out_shape=[
            jax.ShapeDtypeStruct((N, DH), jnp.float32),
            jax.ShapeDtypeStruct((N, DH), jnp.float32),
        ],
    )(ein, w1, w2, wfa, wfb, bf2)


def _edge_body(ha_hbm, hb_hbm, x_hbm, src_hbm, dst_hbm, gd_hbm, wob_hbm,
               zero_hbm, mask_hbm, agg_hbm,
               srcb, dstb, gdb, bufa, bufb, bufx, maskb, dstef, wob,
               aggs, sem0, sem1):
    cid = lax.axis_index("c")
    sid = lax.axis_index("s")
    wid = cid * NS + sid
    ebase = wid * EW

    pltpu.sync_copy(zero_hbm, aggs.at[pl.ds(sid * RPT, RPT)])
    pltpu.sync_copy(wob_hbm, wob)
    plsc.subcore_barrier()

    sems = (sem0, sem1)

    def load_idx(j, p):
        off = ebase + j * K
        pltpu.sync_copy(src_hbm.at[pl.ds(off, K)], srcb.at[p])
        pltpu.sync_copy(dst_hbm.at[pl.ds(off, K)], dstb.at[p])
        pltpu.sync_copy(gd_hbm.at[pl.ds(off, K)], gdb.at[p])

    def fire(p):
        sem = sems[p]
        pltpu.async_copy(ha_hbm.at[srcb.at[p]], bufa.at[p], sem)
        pltpu.async_copy(hb_hbm.at[dstb.at[p]], bufb.at[p], sem)
        pltpu.async_copy(x_hbm.at[srcb.at[p]], bufx.at[p], sem)

    def drain(p):
        sem = sems[p]
        pltpu.make_async_copy(ha_hbm.at[srcb.at[p]], bufa.at[p], sem).wait()
        pltpu.make_async_copy(hb_hbm.at[dstb.at[p]], bufb.at[p], sem).wait()
        pltpu.make_async_copy(x_hbm.at[srcb.at[p]], bufx.at[p], sem).wait()

    def compute_store(j, p):
        ba = bufa.at[p]
        bb = bufb.at[p]

        def group(g, carry):
            rows = lax.iota(jnp.int32, 16) + g * 16
            acc = gdb[p, pl.ds(g * 16, 16)]
            for d in range(DH):
                dvec = jnp.full((16,), d, jnp.int32)
                a = plsc.load_gather(ba, [rows, dvec])
                b = plsc.load_gather(bb, [rows, dvec])
                acc = acc + jnp.maximum(a + b, 0.0) * wob[d, :]
            keep = acc >= 0.0
            dstv = dstb[p, pl.ds(g * 16, 16)]
            maskb[pl.ds(g * 16, 16)] = jnp.where(keep, 1.0, 0.0)
            dstef[pl.ds(g * 16, 16)] = jnp.where(
                keep, dstv, jnp.full((16,), DUMMY, jnp.int32))
            return carry

        lax.fori_loop(0, G, group, 0)
        off = ebase + j * K
        pltpu.sync_copy(maskb, mask_hbm.at[pl.ds(off, K)])
        pltpu.sync_copy(bufx.at[p], aggs.at[dstef], add=True)

    load_idx(0, 0)
    fire(0)

    def dbl(i, carry):
        j0 = 2 * i
        load_idx(j0 + 1, 1)
        fire(1)
        drain(0)
        compute_store(j0, 0)
        load_idx(j0 + 2, 0)
        fire(0)
        drain(1)
        compute_store(j0 + 1, 1)
        return carry

    lax.fori_loop(0, (NCH - 1) // 2, dbl, 0)
    drain(0)
    compute_store(NCH - 1, 0)

    plsc.subcore_barrier()
    pltpu.sync_copy(aggs.at[pl.ds(sid * RPT, RPT)],
                    agg_hbm.at[cid, pl.ds(sid * RPT, RPT)])


def _edge_sc(ha, hb, x2, src, dst, gd, wob, zero):
    mesh = plsc.VectorSubcoreMesh(core_axis_name="c", subcore_axis_name="s")
    fn = pl.kernel(
        _edge_body,
        out_type=[
            jax.ShapeDtypeStruct((E,), jnp.float32),
            jax.ShapeDtypeStruct((NC, AGGR, B * TIN), jnp.float32),
        ],
        mesh=mesh,
        compiler_params=pltpu.CompilerParams(
            needs_layout_passes=False, use_tc_tiling_on_sc=False),
        scratch_types=[
            pltpu.VMEM((2, K), jnp.int32),
            pltpu.VMEM((2, K), jnp.int32),
            pltpu.VMEM((2, K), jnp.float32),
            pltpu.VMEM((2, K, DH), jnp.float32),
            pltpu.VMEM((2, K, DH), jnp.float32),
            pltpu.VMEM((2, K, B * TIN), jnp.float32),
            pltpu.VMEM((K,), jnp.float32),
            pltpu.VMEM((K,), jnp.int32),
            pltpu.VMEM((DH, 16), jnp.float32),
            pltpu.VMEM_SHARED((AGGR, B * TIN), jnp.float32),
            pltpu.SemaphoreType.DMA,
            pltpu.SemaphoreType.DMA,
        ],
    )
    return fn(ha, hb, x2, src, dst, gd, wob, zero)


def _gru_body(x2_ref, agg_ref, wx_ref, bx_ref, wh_ref, wdec_ref, out_ref):
    blk = x2_ref.shape[0]
    wx0 = wx_ref[0:1, :]
    wx1 = wx_ref[1:2, :]
    bx = bx_ref[...]
    wh = wh_ref[...]
    x2 = x2_ref[...]
    agg = agg_ref[0] + agg_ref[1]

    for b in range(B):
        h = jnp.zeros((blk, DH), jnp.float32)
        for t in range(TIN):
            c = b * TIN + t
            xt = x2[:, c:c + 1]
            at = agg[:, c:c + 1]
            xg = xt * wx0 + at * wx1 + bx
            hg = jnp.dot(h, wh, preferred_element_type=jnp.float32)
            z = jax.nn.sigmoid(xg[:, :DH] + hg[:, :DH])
            r = jax.nn.sigmoid(xg[:, DH:2 * DH] + hg[:, DH:2 * DH])
            n = jnp.tanh(xg[:, 2 * DH:] + r * hg[:, 2 * DH:])
            h = (1.0 - z) * n + z * h
        out = jnp.dot(h, wdec_ref[...], preferred_element_type=jnp.float32)
        out_ref[:, b * TOUT:(b + 1) * TOUT] = out


def _gru(x2, agg2, wx, bx2, wh, wdec):
    grid = pl.cdiv(N, BLK_C)
    return pl.pallas_call(
        _gru_body,
        grid=(grid,),
        in_specs=[
            pl.BlockSpec((BLK_C, B * TIN), lambda i: (i, 0)),
            pl.BlockSpec((NC, BLK_C, B * TIN), lambda i: (0, i, 0)),
            pl.BlockSpec((2, 3 * DH), lambda i: (0, 0)),
            pl.BlockSpec((1, 3 * DH), lambda i: (0, 0)),
            pl.BlockSpec((DH, 3 * DH), lambda i: (0, 0)),
            pl.BlockSpec((DH, TOUT), lambda i: (0, 0)),
        ],
        out_specs=pl.BlockSpec((BLK_C, B * TOUT), lambda i: (i, 0)),
        out_shape=jax.ShapeDtypeStruct((N, B * TOUT), jnp.float32),
    )(x2, agg2, wx, bx2, wh, wdec)


def kernel(inputs, targets, entire_inputs, edge_index, W1, W2, Wf, bf, Wo,
           Wx, bx, Wh, Wdec):
    src = edge_index[0].astype(jnp.int32)
    dst = edge_index[1].astype(jnp.int32)

    u = jax.random.uniform(jax.random.key(42), (E, 2),
                           minval=1e-6, maxval=1.0 - 1e-6)
    g = -jnp.log(-jnp.log(u))
    gd = g[:, 0] - g[:, 1]

    wod = Wo[:, 0] - Wo[:, 1]
    wob = jnp.tile(wod[:, None], (1, 16))
    x2 = jnp.transpose(inputs[:, :, :, 0], (2, 0, 1)).reshape(N, B * TIN)
    zero = jnp.zeros((RPT, B * TIN), jnp.float32)

    ha, hb = _encoder(entire_inputs, W1, W2, Wf[:128], Wf[128:],
                      bf.reshape(1, DH))
    mask, agg2 = _edge_sc(ha, hb, x2, src, dst, gd, wob, zero)
    out48 = _gru(x2, agg2, Wx, bx.reshape(1, 3 * DH), Wh, Wdec)
    outputs = out48.reshape(N, B, TOUT).transpose(1, 2, 0)[..., None]
    return (mask, outputs)


# speedup vs baseline: 17.1922x; 1.1089x over previous
import functools

import jax
import jax.numpy as jnp
from jax import lax
from jax.experimental import pallas as pl
from jax.experimental.pallas import tpu as pltpu
from jax.experimental.pallas import tpu_sc as plsc

N = 10000
E = 320000
B = 4
TIN = 12
TOUT = 12
DH = 64

NC = 2
NS = 16
NW = NC * NS
EW = E // NW
K = 128
NF = EW // K
KT = EW - NF * K
G = K // 16
DUMMY = N
AGGR = 10240
RPT = AGGR // NS

BLK_A = 256
BLK_C = 256


def _enc_body(ein_ref, w1_ref, w2_ref, wfa_ref, wfb_ref, bf_ref,
              ha_ref, hb_ref):
    h1 = jnp.maximum(
        jnp.dot(ein_ref[...], w1_ref[...],
                preferred_element_type=jnp.float32), 0.0)
    h = jnp.maximum(
        jnp.dot(h1, w2_ref[...], preferred_element_type=jnp.float32), 0.0)
    ha_ref[...] = (
        jnp.dot(h, wfa_ref[...], preferred_element_type=jnp.float32)
        + bf_ref[...])
    hb_ref[...] = jnp.dot(h, wfb_ref[...],
                          preferred_element_type=jnp.float32)


def _encoder(ein, w1, w2, wfa, wfb, bf2):
    t = ein.shape[1]
    grid = pl.cdiv(N, BLK_A)
    return pl.pallas_call(
        _enc_body,
        grid=(grid,),
        in_specs=[
            pl.BlockSpec((BLK_A, t), lambda i: (i, 0)),
            pl.BlockSpec((t, 256), lambda i: (0, 0)),
            pl.BlockSpec((256, 128), lambda i: (0, 0)),
            pl.BlockSpec((128, DH), lambda i: (0, 0)),
            pl.BlockSpec((128, DH), lambda i: (0, 0)),
            pl.BlockSpec((1, DH), lambda i: (0, 0)),
        ],
        out_specs=[
            pl.BlockSpec((BLK_A, DH), lambda i: (i, 0)),
            pl.BlockSpec((BLK_A, DH), lambda i: (i, 0)),
        ],
        out_shape=[
            jax.ShapeDtypeStruct((N, DH), jnp.float32),
            jax.ShapeDtypeStruct((N, DH), jnp.float32),
        ],
    )(ein, w1, w2, wfa, wfb, bf2)


def _edge_body(ha_hbm, hb_hbm, x_hbm, src_hbm, dst_hbm, gd_hbm, wob_hbm,
               zero_hbm, mask_hbm, agg_hbm,
               srcall, dstall, gdall, maskall, bufa, bufb, bufx, dstef,
               ta, tb, tx, tdst, wob, aggs, sem0, sem1):
    cid = lax.axis_index("c")
    sid = lax.axis_index("s")
    wid = cid * NS + sid
    ebase = wid * EW

    pltpu.sync_copy(zero_hbm, aggs.at[pl.ds(sid * RPT, RPT)])
    pltpu.sync_copy(wob_hbm, wob)
    pltpu.sync_copy(src_hbm.at[pl.ds(ebase, EW)], srcall)
    pltpu.sync_copy(dst_hbm.at[pl.ds(ebase, EW)], dstall)
    pltpu.sync_copy(gd_hbm.at[pl.ds(ebase, EW)], gdall)
    plsc.subcore_barrier()

    sems = (sem0, sem1)

    def fire(j, p):
        sem = sems[p]
        soff = srcall.at[pl.ds(j * K, K)]
        doff = dstall.at[pl.ds(j * K, K)]
        pltpu.async_copy(ha_hbm.at[soff], bufa.at[p], sem)
        pltpu.async_copy(hb_hbm.at[doff], bufb.at[p], sem)
        pltpu.async_copy(x_hbm.at[soff], bufx.at[p], sem)

    def drain(j, p):
        sem = sems[p]
        soff = srcall.at[pl.ds(j * K, K)]
        doff = dstall.at[pl.ds(j * K, K)]
        pltpu.make_async_copy(ha_hbm.at[soff], bufa.at[p], sem).wait()
        pltpu.make_async_copy(hb_hbm.at[doff], bufb.at[p], sem).wait()
        pltpu.make_async_copy(x_hbm.at[soff], bufx.at[p], sem).wait()

    def score_group(ba, bb, ebos, g):
        rows = lax.iota(jnp.int32, 16) + g * 16
        acc = gdall[pl.ds(ebos + g * 16, 16)]
        for d in range(DH):
            dvec = jnp.full((16,), d, jnp.int32)
            a = plsc.load_gather(ba, [rows, dvec])
            b = plsc.load_gather(bb, [rows, dvec])
            acc = acc + jnp.maximum(a + b, 0.0) * wob[d, :]
        keep = acc >= 0.0
        dstv = dstall[pl.ds(ebos + g * 16, 16)]
        maskall[pl.ds(ebos + g * 16, 16)] = jnp.where(keep, 1.0, 0.0)
        return jnp.where(keep, dstv, jnp.full((16,), DUMMY, jnp.int32))

    def compute_store(j, p):
        ba = bufa.at[p]
        bb = bufb.at[p]

        def group(g, carry):
            dstef[p, pl.ds(g * 16, 16)] = score_group(ba, bb, j * K, g)
            return carry

        lax.fori_loop(0, G, group, 0)
        pltpu.sync_copy(bufx.at[p], aggs.at[dstef.at[p]], add=True)

    fire(0, 0)

    def dbl(i, carry):
        j0 = 2 * i
        fire(j0 + 1, 1)
        drain(j0, 0)
        compute_store(j0, 0)
        fire(j0 + 2, 0)
        drain(j0 + 1, 1)
        compute_store(j0 + 1, 1)
        return carry

    lax.fori_loop(0, NF // 2 - 1, dbl, 0)
    fire(NF - 1, 1)
    drain(NF - 2, 0)
    compute_store(NF - 2, 0)
    drain(NF - 1, 1)
    compute_store(NF - 1, 1)

    tsoff = srcall.at[pl.ds(NF * K, KT)]
    tdoff = dstall.at[pl.ds(NF * K, KT)]
    pltpu.async_copy(ha_hbm.at[tsoff], ta, sem0)
    pltpu.async_copy(hb_hbm.at[tdoff], tb, sem0)
    pltpu.async_copy(x_hbm.at[tsoff], tx, sem0)
    pltpu.make_async_copy(ha_hbm.at[tsoff], ta, sem0).wait()
    pltpu.make_async_copy(hb_hbm.at[tdoff], tb, sem0).wait()
    pltpu.make_async_copy(x_hbm.at[tsoff], tx, sem0).wait()
    tdst[...] = score_group(ta, tb, NF * K, 0)
    pltpu.sync_copy(tx, aggs.at[tdst], add=True)

    pltpu.sync_copy(maskall, mask_hbm.at[pl.ds(ebase, EW)])
    plsc.subcore_barrier()
    pltpu.sync_copy(aggs.at[pl.ds(sid * RPT, RPT)],
                    agg_hbm.at[cid, pl.ds(sid * RPT, RPT)])


def _edge_sc(ha, hb, x2, src, dst, gd, wob, zero):
    mesh = plsc.VectorSubcoreMesh(core_axis_name="c", subcore_axis_name="s")
    fn = pl.kernel(
        _edge_body,
        out_type=[
            jax.ShapeDtypeStruct((E,), jnp.float32),
            jax.ShapeDtypeStruct((NC, AGGR, B * TIN), jnp.float32),
        ],
        mesh=mesh,
        compiler_params=pltpu.CompilerParams(
            needs_layout_passes=False, use_tc_tiling_on_sc=False),
        scratch_types=[
            pltpu.VMEM((EW,), jnp.int32),
            pltpu.VMEM((EW,), jnp.int32),
            pltpu.VMEM((EW,), jnp.float32),
            pltpu.VMEM((EW,), jnp.float32),
            pltpu.VMEM((2, K, DH), jnp.float32),
            pltpu.VMEM((2, K, DH), jnp.float32),
            pltpu.VMEM((2, K, B * TIN), jnp.float32),
            pltpu.VMEM((2, K), jnp.int32),
            pltpu.VMEM((KT, DH), jnp.float32),
            pltpu.VMEM((KT, DH), jnp.float32),
            pltpu.VMEM((KT, B * TIN), jnp.float32),
            pltpu.VMEM((KT,), jnp.int32),
            pltpu.VMEM((DH, 16), jnp.float32),
            pltpu.VMEM_SHARED((AGGR, B * TIN), jnp.float32),
            pltpu.SemaphoreType.DMA,
            pltpu.SemaphoreType.DMA,
        ],
    )
    return fn(ha, hb, x2, src, dst, gd, wob, zero)


def _gru_body(x2_ref, agg_ref, wx_ref, bx_ref, wh_ref, wdec_ref, out_ref):
    blk = x2_ref.shape[0]
    wx0 = wx_ref[0:1, :]
    wx1 = wx_ref[1:2, :]
    bx = bx_ref[...]
    wh = wh_ref[...]
    x2 = x2_ref[...]
    agg = agg_ref[0] + agg_ref[1]

    for b in range(B):
        h = jnp.zeros((blk, DH), jnp.float32)
        for t in range(TIN):
            c = b * TIN + t
            xt = x2[:, c:c + 1]
            at = agg[:, c:c + 1]
            xg = xt * wx0 + at * wx1 + bx
            hg = jnp.dot(h, wh, preferred_element_type=jnp.float32)
            z = jax.nn.sigmoid(xg[:, :DH] + hg[:, :DH])
            r = jax.nn.sigmoid(xg[:, DH:2 * DH] + hg[:, DH:2 * DH])
            n = jnp.tanh(xg[:, 2 * DH:] + r * hg[:, 2 * DH:])
            h = (1.0 - z) * n + z * h
        out = jnp.dot(h, wdec_ref[...], preferred_element_type=jnp.float32)
        out_ref[:, b * TOUT:(b + 1) * TOUT] = out


def _gru(x2, agg2, wx, bx2, wh, wdec):
    grid = pl.cdiv(N, BLK_C)
    return pl.pallas_call(
        _gru_body,
        grid=(grid,),
        in_specs=[
            pl.BlockSpec((BLK_C, B * TIN), lambda i: (i, 0)),
            pl.BlockSpec((NC, BLK_C, B * TIN), lambda i: (0, i, 0)),
            pl.BlockSpec((2, 3 * DH), lambda i: (0, 0)),
            pl.BlockSpec((1, 3 * DH), lambda i: (0, 0)),
            pl.BlockSpec((DH, 3 * DH), lambda i: (0, 0)),
            pl.BlockSpec((DH, TOUT), lambda i: (0, 0)),
        ],
        out_specs=pl.BlockSpec((BLK_C, B * TOUT), lambda i: (i, 0)),
        out_shape=jax.ShapeDtypeStruct((N, B * TOUT), jnp.float32),
    )(x2, agg2, wx, bx2, wh, wdec)


def kernel(inputs, targets, entire_inputs, edge_index, W1, W2, Wf, bf, Wo,
           Wx, bx, Wh, Wdec):
    src = edge_index[0].astype(jnp.int32)
    dst = edge_index[1].astype(jnp.int32)

    u = jax.random.uniform(jax.random.key(42), (E, 2),
                           minval=1e-6, maxval=1.0 - 1e-6)
    g = -jnp.log(-jnp.log(u))
    gd = g[:, 0] - g[:, 1]

    wod = Wo[:, 0] - Wo[:, 1]
    wob = jnp.tile(wod[:, None], (1, 16))
    x2 = jnp.transpose(inputs[:, :, :, 0], (2, 0, 1)).reshape(N, B * TIN)
    zero = jnp.zeros((RPT, B * TIN), jnp.float32)

    ha, hb = _encoder(entire_inputs, W1, W2, Wf[:128], Wf[128:],
                      bf.reshape(1, DH))
    mask, agg2 = _edge_sc(ha, hb, x2, src, dst, gd, wob, zero)
    out48 = _gru(x2, agg2, Wx, bx.reshape(1, 3 * DH), Wh, Wdec)
    outputs = out48.reshape(N, B, TOUT).transpose(1, 2, 0)[..., None]
    return (mask, outputs)


# speedup vs baseline: 17.2852x; 1.0054x over previous
import functools

import jax
import jax.numpy as jnp
from jax import lax
from jax.experimental import pallas as pl
from jax.experimental.pallas import tpu as pltpu
from jax.experimental.pallas import tpu_sc as plsc

N = 10000
E = 320000
B = 4
TIN = 12
TOUT = 12
DH = 64

NC = 2
NS = 16
NW = NC * NS
EW = E // NW
K = 128
NF = EW // K
KT = EW - NF * K
G = K // 16
DUMMY = N
AGGR = 10240
RPT = AGGR // NS

BLK_A = 256
BLK_C = 256


def _enc_body(ein_ref, w1_ref, w2_ref, wfa_ref, wfb_ref, bf_ref,
              ha_ref, hb_ref):
    h1 = jnp.maximum(
        jnp.dot(ein_ref[...], w1_ref[...],
                preferred_element_type=jnp.float32), 0.0)
    h = jnp.maximum(
        jnp.dot(h1, w2_ref[...], preferred_element_type=jnp.float32), 0.0)
    ha_ref[...] = (
        jnp.dot(h, wfa_ref[...], preferred_element_type=jnp.float32)
        + bf_ref[...])
    hb_ref[...] = jnp.dot(h, wfb_ref[...],
                          preferred_element_type=jnp.float32)


def _encoder(ein, w1, w2, wfa, wfb, bf2):
    t = ein.shape[1]
    grid = pl.cdiv(N, BLK_A)
    return pl.pallas_call(
        _enc_body,
        grid=(grid,),
        in_specs=[
            pl.BlockSpec((BLK_A, t), lambda i: (i, 0)),
            pl.BlockSpec((t, 256), lambda i: (0, 0)),
            pl.BlockSpec((256, 128), lambda i: (0, 0)),
            pl.BlockSpec((128, DH), lambda i: (0, 0)),
            pl.BlockSpec((128, DH), lambda i: (0, 0)),
            pl.BlockSpec((1, DH), lambda i: (0, 0)),
        ],
        out_specs=[
            pl.BlockSpec((BLK_A, DH), lambda i: (i, 0)),
            pl.BlockSpec((BLK_A, DH), lambda i: (i, 0)),
        ],
        out_shape=[
            jax.ShapeDtypeStruct((N, DH), jnp.float32),
            jax.ShapeDtypeStruct((N, DH), jnp.float32),
        ],
    )(ein, w1, w2, wfa, wfb, bf2)


def _edge_body(ha_hbm, hb_hbm, x_hbm, src_hbm, dst_hbm, gd_hbm, wob_hbm,
               zero_hbm, mask_hbm, agg_hbm,
               srcall, dstall, gdall, maskall, bufa, bufb, bufx, dstef,
               ta, tb, tx, tdst, wob, aggs, sem0, sem1):
    cid = lax.axis_index("c")
    sid = lax.axis_index("s")
    wid = cid * NS + sid
    ebase = wid * EW

    pltpu.sync_copy(zero_hbm, aggs.at[pl.ds(sid * RPT, RPT)])
    pltpu.sync_copy(wob_hbm, wob)
    pltpu.sync_copy(src_hbm.at[pl.ds(ebase, EW)], srcall)
    pltpu.sync_copy(dst_hbm.at[pl.ds(ebase, EW)], dstall)
    pltpu.sync_copy(gd_hbm.at[pl.ds(ebase, EW)], gdall)
    plsc.subcore_barrier()

    sems = (sem0, sem1)

    def fire(j, p):
        sem = sems[p]
        soff = srcall.at[pl.ds(j * K, K)]
        doff = dstall.at[pl.ds(j * K, K)]
        pltpu.async_copy(ha_hbm.at[soff], bufa.at[p], sem)
        pltpu.async_copy(hb_hbm.at[doff], bufb.at[p], sem)
        pltpu.async_copy(x_hbm.at[soff], bufx.at[p], sem)

    def drain(j, p):
        sem = sems[p]
        soff = srcall.at[pl.ds(j * K, K)]
        doff = dstall.at[pl.ds(j * K, K)]
        pltpu.make_async_copy(ha_hbm.at[soff], bufa.at[p], sem).wait()
        pltpu.make_async_copy(hb_hbm.at[doff], bufb.at[p], sem).wait()
        pltpu.make_async_copy(x_hbm.at[soff], bufx.at[p], sem).wait()

    def score_group(ba, bb, ebos, g):
        rows = lax.iota(jnp.int32, 16) + g * 16
        part = [gdall[pl.ds(ebos + g * 16, 16)]] + [
            jnp.zeros((16,), jnp.float32) for _ in range(3)]
        for d in range(DH):
            dvec = jnp.full((16,), d, jnp.int32)
            a = plsc.load_gather(ba, [rows, dvec])
            b = plsc.load_gather(bb, [rows, dvec])
            part[d % 4] = part[d % 4] + jnp.maximum(a + b, 0.0) * wob[d, :]
        acc = (part[0] + part[1]) + (part[2] + part[3])
        keep = acc >= 0.0
        dstv = dstall[pl.ds(ebos + g * 16, 16)]
        maskall[pl.ds(ebos + g * 16, 16)] = jnp.where(keep, 1.0, 0.0)
        return jnp.where(keep, dstv, jnp.full((16,), DUMMY, jnp.int32))

    def compute_store(j, p):
        ba = bufa.at[p]
        bb = bufb.at[p]

        def group(g, carry):
            dstef[p, pl.ds(g * 16, 16)] = score_group(ba, bb, j * K, g)
            return carry

        lax.fori_loop(0, G, group, 0)
        pltpu.sync_copy(bufx.at[p], aggs.at[dstef.at[p]], add=True)

    fire(0, 0)

    def dbl(i, carry):
        j0 = 2 * i
        fire(j0 + 1, 1)
        drain(j0, 0)
        compute_store(j0, 0)
        fire(j0 + 2, 0)
        drain(j0 + 1, 1)
        compute_store(j0 + 1, 1)
        return carry

    lax.fori_loop(0, NF // 2 - 1, dbl, 0)
    fire(NF - 1, 1)
    drain(NF - 2, 0)
    compute_store(NF - 2, 0)
    drain(NF - 1, 1)
    compute_store(NF - 1, 1)

    tsoff = srcall.at[pl.ds(NF * K, KT)]
    tdoff = dstall.at[pl.ds(NF * K, KT)]
    pltpu.async_copy(ha_hbm.at[tsoff], ta, sem0)
    pltpu.async_copy(hb_hbm.at[tdoff], tb, sem0)
    pltpu.async_copy(x_hbm.at[tsoff], tx, sem0)
    pltpu.make_async_copy(ha_hbm.at[tsoff], ta, sem0).wait()
    pltpu.make_async_copy(hb_hbm.at[tdoff], tb, sem0).wait()
    pltpu.make_async_copy(x_hbm.at[tsoff], tx, sem0).wait()
    tdst[...] = score_group(ta, tb, NF * K, 0)
    pltpu.sync_copy(tx, aggs.at[tdst], add=True)

    pltpu.sync_copy(maskall, mask_hbm.at[pl.ds(ebase, EW)])
    plsc.subcore_barrier()
    pltpu.sync_copy(aggs.at[pl.ds(sid * RPT, RPT)],
                    agg_hbm.at[cid, pl.ds(sid * RPT, RPT)])


def _edge_sc(ha, hb, x2, src, dst, gd, wob, zero):
    mesh = plsc.VectorSubcoreMesh(core_axis_name="c", subcore_axis_name="s")
    fn = pl.kernel(
        _edge_body,
        out_type=[
            jax.ShapeDtypeStruct((E,), jnp.float32),
            jax.ShapeDtypeStruct((NC, AGGR, B * TIN), jnp.float32),
        ],
        mesh=mesh,
        compiler_params=pltpu.CompilerParams(
            needs_layout_passes=False, use_tc_tiling_on_sc=False),
        scratch_types=[
            pltpu.VMEM((EW,), jnp.int32),
            pltpu.VMEM((EW,), jnp.int32),
            pltpu.VMEM((EW,), jnp.float32),
            pltpu.VMEM((EW,), jnp.float32),
            pltpu.VMEM((2, K, DH), jnp.float32),
            pltpu.VMEM((2, K, DH), jnp.float32),
            pltpu.VMEM((2, K, B * TIN), jnp.float32),
            pltpu.VMEM((2, K), jnp.int32),
            pltpu.VMEM((KT, DH), jnp.float32),
            pltpu.VMEM((KT, DH), jnp.float32),
            pltpu.VMEM((KT, B * TIN), jnp.float32),
            pltpu.VMEM((KT,), jnp.int32),
            pltpu.VMEM((DH, 16), jnp.float32),
            pltpu.VMEM_SHARED((AGGR, B * TIN), jnp.float32),
            pltpu.SemaphoreType.DMA,
            pltpu.SemaphoreType.DMA,
        ],
    )
    return fn(ha, hb, x2, src, dst, gd, wob, zero)


def _gru_body(x2_ref, agg_ref, wx_ref, bx_ref, wh_ref, wdec_ref, out_ref):
    blk = x2_ref.shape[0]
    wx0 = wx_ref[0:1, :]
    wx1 = wx_ref[1:2, :]
    bx = bx_ref[...]
    wh = wh_ref[...]
    x2 = x2_ref[...]
    agg = agg_ref[0] + agg_ref[1]

    for b in range(B):
        h = jnp.zeros((blk, DH), jnp.float32)
        for t in range(TIN):
            c = b * TIN + t
            xt = x2[:, c:c + 1]
            at = agg[:, c:c + 1]
            xg = xt * wx0 + at * wx1 + bx
            hg = jnp.dot(h, wh, preferred_element_type=jnp.float32)
            z = jax.nn.sigmoid(xg[:, :DH] + hg[:, :DH])
            r = jax.nn.sigmoid(xg[:, DH:2 * DH] + hg[:, DH:2 * DH])
            n = jnp.tanh(xg[:, 2 * DH:] + r * hg[:, 2 * DH:])
            h = (1.0 - z) * n + z * h
        out = jnp.dot(h, wdec_ref[...], preferred_element_type=jnp.float32)
        out_ref[:, b * TOUT:(b + 1) * TOUT] = out


def _gru(x2, agg2, wx, bx2, wh, wdec):
    grid = pl.cdiv(N, BLK_C)
    return pl.pallas_call(
        _gru_body,
        grid=(grid,),
        in_specs=[
            pl.BlockSpec((BLK_C, B * TIN), lambda i: (i, 0)),
            pl.BlockSpec((NC, BLK_C, B * TIN), lambda i: (0, i, 0)),
            pl.BlockSpec((2, 3 * DH), lambda i: (0, 0)),
            pl.BlockSpec((1, 3 * DH), lambda i: (0, 0)),
            pl.BlockSpec((DH, 3 * DH), lambda i: (0, 0)),
            pl.BlockSpec((DH, TOUT), lambda i: (0, 0)),
        ],
        out_specs=pl.BlockSpec((BLK_C, B * TOUT), lambda i: (i, 0)),
        out_shape=jax.ShapeDtypeStruct((N, B * TOUT), jnp.float32),
    )(x2, agg2, wx, bx2, wh, wdec)


def kernel(inputs, targets, entire_inputs, edge_index, W1, W2, Wf, bf, Wo,
           Wx, bx, Wh, Wdec):
    src = edge_index[0].astype(jnp.int32)
    dst = edge_index[1].astype(jnp.int32)

    u = jax.random.uniform(jax.random.key(42), (E, 2),
                           minval=1e-6, maxval=1.0 - 1e-6)
    g = -jnp.log(-jnp.log(u))
    gd = g[:, 0] - g[:, 1]

    wod = Wo[:, 0] - Wo[:, 1]
    wob = jnp.tile(wod[:, None], (1, 16))
    x2 = jnp.transpose(inputs[:, :, :, 0], (2, 0, 1)).reshape(N, B * TIN)
    zero = jnp.zeros((RPT, B * TIN), jnp.float32)

    ha, hb = _encoder(entire_inputs, W1, W2, Wf[:128], Wf[128:],
                      bf.reshape(1, DH))
    mask, agg2 = _edge_sc(ha, hb, x2, src, dst, gd, wob, zero)
    out48 = _gru(x2, agg2, Wx, bx.reshape(1, 3 * DH), Wh, Wdec)
    outputs = out48.reshape(N, B, TOUT).transpose(1, 2, 0)[..., None]
    return (mask, outputs)


# speedup vs baseline: 28.1682x; 1.6296x over previous
import functools

import jax
import jax.numpy as jnp
from jax import lax
from jax.experimental import pallas as pl
from jax.experimental.pallas import tpu as pltpu
from jax.experimental.pallas import tpu_sc as plsc

N = 10000
E = 320000
B = 4
TIN = 12
TOUT = 12
DH = 64

NC = 2
NS = 16
NW = NC * NS
EW = E // NW
K = 128
NF = EW // K
KT = EW - NF * K
G = K // 16
DUMMY = N
AGGR = 10240
RPT = AGGR // NS

BLK_A = 256
BLK_C = 256


def _enc_body(ein_ref, w1_ref, w2_ref, wfa_ref, wfb_ref, bf_ref,
              ha_ref, hb_ref):
    h1 = jnp.maximum(
        jnp.dot(ein_ref[...], w1_ref[...],
                preferred_element_type=jnp.float32), 0.0)
    h = jnp.maximum(
        jnp.dot(h1, w2_ref[...], preferred_element_type=jnp.float32), 0.0)
    ha_ref[...] = (
        jnp.dot(h, wfa_ref[...], preferred_element_type=jnp.float32)
        + bf_ref[...])
    hb_ref[...] = jnp.dot(h, wfb_ref[...],
                          preferred_element_type=jnp.float32)


def _encoder(ein, w1, w2, wfa, wfb, bf2):
    t = ein.shape[1]
    grid = pl.cdiv(N, BLK_A)
    return pl.pallas_call(
        _enc_body,
        grid=(grid,),
        in_specs=[
            pl.BlockSpec((BLK_A, t), lambda i: (i, 0)),
            pl.BlockSpec((t, 256), lambda i: (0, 0)),
            pl.BlockSpec((256, 128), lambda i: (0, 0)),
            pl.BlockSpec((128, DH), lambda i: (0, 0)),
            pl.BlockSpec((128, DH), lambda i: (0, 0)),
            pl.BlockSpec((1, DH), lambda i: (0, 0)),
        ],
        out_specs=[
            pl.BlockSpec((BLK_A, DH), lambda i: (i, 0)),
            pl.BlockSpec((BLK_A, DH), lambda i: (i, 0)),
        ],
        out_shape=[
            jax.ShapeDtypeStruct((N, DH), jnp.float32),
            jax.ShapeDtypeStruct((N, DH), jnp.float32),
        ],
    )(ein, w1, w2, wfa, wfb, bf2)


def _edge_body(ha_hbm, hb_hbm, x_hbm, src_hbm, dst_hbm, gd_hbm, wob_hbm,
               zero_hbm, mask_hbm, agg_hbm,
               srcall, dstall, gdall, maskall, bufa, bufb, bufx, dstef,
               ta, tb, tx, tdst, wob, aggs, sem0, sem1):
    cid = lax.axis_index("c")
    sid = lax.axis_index("s")
    wid = cid * NS + sid
    ebase = wid * EW

    pltpu.sync_copy(zero_hbm, aggs.at[pl.ds(sid * RPT, RPT)])
    pltpu.sync_copy(wob_hbm, wob)
    pltpu.sync_copy(src_hbm.at[pl.ds(ebase, EW)], srcall)
    pltpu.sync_copy(dst_hbm.at[pl.ds(ebase, EW)], dstall)
    pltpu.sync_copy(gd_hbm.at[pl.ds(ebase, EW)], gdall)
    plsc.subcore_barrier()

    sems = (sem0, sem1)

    def fire(j, p):
        sem = sems[p]
        soff = srcall.at[pl.ds(j * K, K)]
        doff = dstall.at[pl.ds(j * K, K)]
        pltpu.async_copy(ha_hbm.at[soff], bufa.at[p], sem)
        pltpu.async_copy(hb_hbm.at[doff], bufb.at[p], sem)
        pltpu.async_copy(x_hbm.at[soff], bufx.at[p], sem)

    def drain(j, p):
        sem = sems[p]
        soff = srcall.at[pl.ds(j * K, K)]
        doff = dstall.at[pl.ds(j * K, K)]
        pltpu.make_async_copy(ha_hbm.at[soff], bufa.at[p], sem).wait()
        pltpu.make_async_copy(hb_hbm.at[doff], bufb.at[p], sem).wait()
        pltpu.make_async_copy(x_hbm.at[soff], bufx.at[p], sem).wait()

    wc = [wob[pl.ds(16 * c, 16)] for c in range(DH // 16)]

    def score_group(ba, bb, ebos, g):
        base = ebos + g * 16
        gdv = gdall[pl.ds(base, 16)]
        dstv = dstall[pl.ds(base, 16)]
        lanes = lax.iota(jnp.int32, 16)
        tot = gdv
        for i in range(16):
            e = g * 16 + i
            t = []
            for c in range(DH // 16):
                va = ba[e, pl.ds(16 * c, 16)]
                vb = bb[e, pl.ds(16 * c, 16)]
                t.append(jnp.maximum(va + vb, 0.0) * wc[c])
            s = jnp.sum((t[0] + t[1]) + (t[2] + t[3]))
            tot = tot + jnp.where(lanes == i, s, 0.0)
        keep = tot >= 0.0
        maskall[pl.ds(base, 16)] = jnp.where(keep, 1.0, 0.0)
        return jnp.where(keep, dstv, jnp.full((16,), DUMMY, jnp.int32))

    def compute_store(j, p):
        ba = bufa.at[p]
        bb = bufb.at[p]

        def group(g, carry):
            dstef[p, pl.ds(g * 16, 16)] = score_group(ba, bb, j * K, g)
            return carry

        lax.fori_loop(0, G, group, 0)
        pltpu.sync_copy(bufx.at[p], aggs.at[dstef.at[p]], add=True)

    fire(0, 0)

    def dbl(i, carry):
        j0 = 2 * i
        fire(j0 + 1, 1)
        drain(j0, 0)
        compute_store(j0, 0)
        fire(j0 + 2, 0)
        drain(j0 + 1, 1)
        compute_store(j0 + 1, 1)
        return carry

    lax.fori_loop(0, NF // 2 - 1, dbl, 0)
    fire(NF - 1, 1)
    drain(NF - 2, 0)
    compute_store(NF - 2, 0)
    drain(NF - 1, 1)
    compute_store(NF - 1, 1)

    tsoff = srcall.at[pl.ds(NF * K, KT)]
    tdoff = dstall.at[pl.ds(NF * K, KT)]
    pltpu.async_copy(ha_hbm.at[tsoff], ta, sem0)
    pltpu.async_copy(hb_hbm.at[tdoff], tb, sem0)
    pltpu.async_copy(x_hbm.at[tsoff], tx, sem0)
    pltpu.make_async_copy(ha_hbm.at[tsoff], ta, sem0).wait()
    pltpu.make_async_copy(hb_hbm.at[tdoff], tb, sem0).wait()
    pltpu.make_async_copy(x_hbm.at[tsoff], tx, sem0).wait()
    tdst[...] = score_group(ta, tb, NF * K, 0)
    pltpu.sync_copy(tx, aggs.at[tdst], add=True)

    pltpu.sync_copy(maskall, mask_hbm.at[pl.ds(ebase, EW)])
    plsc.subcore_barrier()
    pltpu.sync_copy(aggs.at[pl.ds(sid * RPT, RPT)],
                    agg_hbm.at[cid, pl.ds(sid * RPT, RPT)])


def _edge_sc(ha, hb, x2, src, dst, gd, wob, zero):
    mesh = plsc.VectorSubcoreMesh(core_axis_name="c", subcore_axis_name="s")
    fn = pl.kernel(
        _edge_body,
        out_type=[
            jax.ShapeDtypeStruct((E,), jnp.float32),
            jax.ShapeDtypeStruct((NC, AGGR, B * TIN), jnp.float32),
        ],
        mesh=mesh,
        compiler_params=pltpu.CompilerParams(
            needs_layout_passes=False, use_tc_tiling_on_sc=False),
        scratch_types=[
            pltpu.VMEM((EW,), jnp.int32),
            pltpu.VMEM((EW,), jnp.int32),
            pltpu.VMEM((EW,), jnp.float32),
            pltpu.VMEM((EW,), jnp.float32),
            pltpu.VMEM((2, K, DH), jnp.float32),
            pltpu.VMEM((2, K, DH), jnp.float32),
            pltpu.VMEM((2, K, B * TIN), jnp.float32),
            pltpu.VMEM((2, K), jnp.int32),
            pltpu.VMEM((KT, DH), jnp.float32),
            pltpu.VMEM((KT, DH), jnp.float32),
            pltpu.VMEM((KT, B * TIN), jnp.float32),
            pltpu.VMEM((KT,), jnp.int32),
            pltpu.VMEM((DH,), jnp.float32),
            pltpu.VMEM_SHARED((AGGR, B * TIN), jnp.float32),
            pltpu.SemaphoreType.DMA,
            pltpu.SemaphoreType.DMA,
        ],
    )
    return fn(ha, hb, x2, src, dst, gd, wob, zero)


def _gru_body(x2_ref, agg_ref, wx_ref, bx_ref, wh_ref, wdec_ref, out_ref):
    blk = x2_ref.shape[0]
    wx0 = wx_ref[0:1, :]
    wx1 = wx_ref[1:2, :]
    bx = bx_ref[...]
    wh = wh_ref[...]
    x2 = x2_ref[...]
    agg = agg_ref[0] + agg_ref[1]

    for b in range(B):
        h = jnp.zeros((blk, DH), jnp.float32)
        for t in range(TIN):
            c = b * TIN + t
            xt = x2[:, c:c + 1]
            at = agg[:, c:c + 1]
            xg = xt * wx0 + at * wx1 + bx
            hg = jnp.dot(h, wh, preferred_element_type=jnp.float32)
            z = jax.nn.sigmoid(xg[:, :DH] + hg[:, :DH])
            r = jax.nn.sigmoid(xg[:, DH:2 * DH] + hg[:, DH:2 * DH])
            n = jnp.tanh(xg[:, 2 * DH:] + r * hg[:, 2 * DH:])
            h = (1.0 - z) * n + z * h
        out = jnp.dot(h, wdec_ref[...], preferred_element_type=jnp.float32)
        out_ref[:, b * TOUT:(b + 1) * TOUT] = out


def _gru(x2, agg2, wx, bx2, wh, wdec):
    grid = pl.cdiv(N, BLK_C)
    return pl.pallas_call(
        _gru_body,
        grid=(grid,),
        in_specs=[
            pl.BlockSpec((BLK_C, B * TIN), lambda i: (i, 0)),
            pl.BlockSpec((NC, BLK_C, B * TIN), lambda i: (0, i, 0)),
            pl.BlockSpec((2, 3 * DH), lambda i: (0, 0)),
            pl.BlockSpec((1, 3 * DH), lambda i: (0, 0)),
            pl.BlockSpec((DH, 3 * DH), lambda i: (0, 0)),
            pl.BlockSpec((DH, TOUT), lambda i: (0, 0)),
        ],
        out_specs=pl.BlockSpec((BLK_C, B * TOUT), lambda i: (i, 0)),
        out_shape=jax.ShapeDtypeStruct((N, B * TOUT), jnp.float32),
    )(x2, agg2, wx, bx2, wh, wdec)


def kernel(inputs, targets, entire_inputs, edge_index, W1, W2, Wf, bf, Wo,
           Wx, bx, Wh, Wdec):
    src = edge_index[0].astype(jnp.int32)
    dst = edge_index[1].astype(jnp.int32)

    u = jax.random.uniform(jax.random.key(42), (E, 2),
                           minval=1e-6, maxval=1.0 - 1e-6)
    g = -jnp.log(-jnp.log(u))
    gd = g[:, 0] - g[:, 1]

    wob = Wo[:, 0] - Wo[:, 1]
    x2 = jnp.transpose(inputs[:, :, :, 0], (2, 0, 1)).reshape(N, B * TIN)
    zero = jnp.zeros((RPT, B * TIN), jnp.float32)

    ha, hb = _encoder(entire_inputs, W1, W2, Wf[:128], Wf[128:],
                      bf.reshape(1, DH))
    mask, agg2 = _edge_sc(ha, hb, x2, src, dst, gd, wob, zero)
    out48 = _gru(x2, agg2, Wx, bx.reshape(1, 3 * DH), Wh, Wdec)
    outputs = out48.reshape(N, B, TOUT).transpose(1, 2, 0)[..., None]
    return (mask, outputs)


# speedup vs baseline: 36.1895x; 1.2848x over previous
import functools

import jax
import jax.numpy as jnp
from jax import lax
from jax.experimental import pallas as pl
from jax.experimental.pallas import tpu as pltpu
from jax.experimental.pallas import tpu_sc as plsc

N = 10000
E = 320000
B = 4
TIN = 12
TOUT = 12
DH = 64

NC = 2
NS = 16
NW = NC * NS
EW = E // NW
K = 128
NF = EW // K
KT = EW - NF * K
G = K // 16
DUMMY = N
AGGR = 10240
RPT = AGGR // NS

BLK_A = 256
BLK_C = 512


def _enc_body(ein_ref, w1_ref, w2_ref, wfa_ref, wfb_ref, bf_ref,
              ha_ref, hb_ref):
    h1 = jnp.maximum(
        jnp.dot(ein_ref[...], w1_ref[...],
                preferred_element_type=jnp.float32), 0.0)
    h = jnp.maximum(
        jnp.dot(h1, w2_ref[...], preferred_element_type=jnp.float32), 0.0)
    ha_ref[...] = (
        jnp.dot(h, wfa_ref[...], preferred_element_type=jnp.float32)
        + bf_ref[...])
    hb_ref[...] = jnp.dot(h, wfb_ref[...],
                          preferred_element_type=jnp.float32)


def _encoder(ein, w1, w2, wfa, wfb, bf2):
    t = ein.shape[1]
    grid = pl.cdiv(N, BLK_A)
    return pl.pallas_call(
        _enc_body,
        grid=(grid,),
        in_specs=[
            pl.BlockSpec((BLK_A, t), lambda i: (i, 0)),
            pl.BlockSpec((t, 256), lambda i: (0, 0)),
            pl.BlockSpec((256, 128), lambda i: (0, 0)),
            pl.BlockSpec((128, DH), lambda i: (0, 0)),
            pl.BlockSpec((128, DH), lambda i: (0, 0)),
            pl.BlockSpec((1, DH), lambda i: (0, 0)),
        ],
        out_specs=[
            pl.BlockSpec((BLK_A, DH), lambda i: (i, 0)),
            pl.BlockSpec((BLK_A, DH), lambda i: (i, 0)),
        ],
        out_shape=[
            jax.ShapeDtypeStruct((N, DH), jnp.float32),
            jax.ShapeDtypeStruct((N, DH), jnp.float32),
        ],
    )(ein, w1, w2, wfa, wfb, bf2)


def _edge_body(ha_hbm, hb_hbm, x_hbm, src_hbm, dst_hbm, gd_hbm, wob_hbm,
               zero_hbm, mask_hbm, agg_hbm,
               srcall, dstall, gdall, maskall, bufa, bufb, bufx, dstef,
               ta, tb, tx, tdst, wob, aggs, sem0, sem1):
    cid = lax.axis_index("c")
    sid = lax.axis_index("s")
    wid = cid * NS + sid
    ebase = wid * EW

    pltpu.sync_copy(zero_hbm, aggs.at[pl.ds(sid * RPT, RPT)])
    pltpu.sync_copy(wob_hbm, wob)
    pltpu.sync_copy(src_hbm.at[pl.ds(ebase, EW)], srcall)
    pltpu.sync_copy(dst_hbm.at[pl.ds(ebase, EW)], dstall)
    pltpu.sync_copy(gd_hbm.at[pl.ds(ebase, EW)], gdall)
    plsc.subcore_barrier()

    sems = (sem0, sem1)

    def fire(j, p):
        sem = sems[p]
        soff = srcall.at[pl.ds(j * K, K)]
        doff = dstall.at[pl.ds(j * K, K)]
        pltpu.async_copy(ha_hbm.at[soff], bufa.at[p], sem)
        pltpu.async_copy(hb_hbm.at[doff], bufb.at[p], sem)
        pltpu.async_copy(x_hbm.at[soff], bufx.at[p], sem)

    def drain(j, p):
        sem = sems[p]
        soff = srcall.at[pl.ds(j * K, K)]
        doff = dstall.at[pl.ds(j * K, K)]
        pltpu.make_async_copy(ha_hbm.at[soff], bufa.at[p], sem).wait()
        pltpu.make_async_copy(hb_hbm.at[doff], bufb.at[p], sem).wait()
        pltpu.make_async_copy(x_hbm.at[soff], bufx.at[p], sem).wait()

    wc = [wob[pl.ds(16 * c, 16)] for c in range(DH // 16)]

    def score_group(ba, bb, ebos, g):
        base = ebos + g * 16
        gdv = gdall[pl.ds(base, 16)]
        dstv = dstall[pl.ds(base, 16)]
        lanes = lax.iota(jnp.int32, 16)
        tot = gdv
        for i in range(16):
            e = g * 16 + i
            t = []
            for c in range(DH // 16):
                va = ba[e, pl.ds(16 * c, 16)]
                vb = bb[e, pl.ds(16 * c, 16)]
                t.append(jnp.maximum(va + vb, 0.0) * wc[c])
            s = jnp.sum((t[0] + t[1]) + (t[2] + t[3]))
            tot = tot + jnp.where(lanes == i, s, 0.0)
        keep = tot >= 0.0
        maskall[pl.ds(base, 16)] = jnp.where(keep, 1.0, 0.0)
        return jnp.where(keep, dstv, jnp.full((16,), DUMMY, jnp.int32))

    def compute_store(j, p):
        ba = bufa.at[p]
        bb = bufb.at[p]

        def group(g, carry):
            dstef[p, pl.ds(g * 16, 16)] = score_group(ba, bb, j * K, g)
            return carry

        lax.fori_loop(0, G, group, 0)
        pltpu.sync_copy(bufx.at[p], aggs.at[dstef.at[p]], add=True)

    fire(0, 0)

    def dbl(i, carry):
        j0 = 2 * i
        fire(j0 + 1, 1)
        drain(j0, 0)
        compute_store(j0, 0)
        fire(j0 + 2, 0)
        drain(j0 + 1, 1)
        compute_store(j0 + 1, 1)
        return carry

    lax.fori_loop(0, NF // 2 - 1, dbl, 0)
    fire(NF - 1, 1)
    drain(NF - 2, 0)
    compute_store(NF - 2, 0)
    drain(NF - 1, 1)
    compute_store(NF - 1, 1)

    tsoff = srcall.at[pl.ds(NF * K, KT)]
    tdoff = dstall.at[pl.ds(NF * K, KT)]
    pltpu.async_copy(ha_hbm.at[tsoff], ta, sem0)
    pltpu.async_copy(hb_hbm.at[tdoff], tb, sem0)
    pltpu.async_copy(x_hbm.at[tsoff], tx, sem0)
    pltpu.make_async_copy(ha_hbm.at[tsoff], ta, sem0).wait()
    pltpu.make_async_copy(hb_hbm.at[tdoff], tb, sem0).wait()
    pltpu.make_async_copy(x_hbm.at[tsoff], tx, sem0).wait()
    tdst[...] = score_group(ta, tb, NF * K, 0)
    pltpu.sync_copy(tx, aggs.at[tdst], add=True)

    pltpu.sync_copy(maskall, mask_hbm.at[pl.ds(ebase, EW)])
    plsc.subcore_barrier()
    pltpu.sync_copy(aggs.at[pl.ds(sid * RPT, RPT)],
                    agg_hbm.at[cid, pl.ds(sid * RPT, RPT)])


def _edge_sc(ha, hb, x2, src, dst, gd, wob, zero):
    mesh = plsc.VectorSubcoreMesh(core_axis_name="c", subcore_axis_name="s")
    fn = pl.kernel(
        _edge_body,
        out_type=[
            jax.ShapeDtypeStruct((E,), jnp.float32),
            jax.ShapeDtypeStruct((NC, AGGR, B * TIN), jnp.float32),
        ],
        mesh=mesh,
        compiler_params=pltpu.CompilerParams(
            needs_layout_passes=False, use_tc_tiling_on_sc=False),
        scratch_types=[
            pltpu.VMEM((EW,), jnp.int32),
            pltpu.VMEM((EW,), jnp.int32),
            pltpu.VMEM((EW,), jnp.float32),
            pltpu.VMEM((EW,), jnp.float32),
            pltpu.VMEM((2, K, DH), jnp.float32),
            pltpu.VMEM((2, K, DH), jnp.float32),
            pltpu.VMEM((2, K, B * TIN), jnp.float32),
            pltpu.VMEM((2, K), jnp.int32),
            pltpu.VMEM((KT, DH), jnp.float32),
            pltpu.VMEM((KT, DH), jnp.float32),
            pltpu.VMEM((KT, B * TIN), jnp.float32),
            pltpu.VMEM((KT,), jnp.int32),
            pltpu.VMEM((DH,), jnp.float32),
            pltpu.VMEM_SHARED((AGGR, B * TIN), jnp.float32),
            pltpu.SemaphoreType.DMA,
            pltpu.SemaphoreType.DMA,
        ],
    )
    return fn(ha, hb, x2, src, dst, gd, wob, zero)


def _gru_body(x2_ref, agg_ref, wx_ref, bx_ref, wh_ref, wdec_ref, out_ref):
    blk = x2_ref.shape[0]
    wx0 = wx_ref[0:1, :]
    wx1 = wx_ref[1:2, :]
    bx = bx_ref[...]
    wh = wh_ref[...]
    x2 = x2_ref[...]
    agg = agg_ref[0] + agg_ref[1]

    h = jnp.zeros((B * blk, DH), jnp.float32)
    for t in range(TIN):
        parts = []
        for b in range(B):
            c = b * TIN + t
            parts.append(x2[:, c:c + 1] * wx0 + agg[:, c:c + 1] * wx1)
        xg = jnp.concatenate(parts, axis=0) + bx
        hg = jnp.dot(h, wh, preferred_element_type=jnp.float32)
        z = jax.nn.sigmoid(xg[:, :DH] + hg[:, :DH])
        r = jax.nn.sigmoid(xg[:, DH:2 * DH] + hg[:, DH:2 * DH])
        n = jnp.tanh(xg[:, 2 * DH:] + r * hg[:, 2 * DH:])
        h = (1.0 - z) * n + z * h
    out = jnp.dot(h, wdec_ref[...], preferred_element_type=jnp.float32)
    for b in range(B):
        out_ref[:, b * TOUT:(b + 1) * TOUT] = out[b * blk:(b + 1) * blk, :]


def _gru(x2, agg2, wx, bx2, wh, wdec):
    grid = pl.cdiv(N, BLK_C)
    return pl.pallas_call(
        _gru_body,
        grid=(grid,),
        in_specs=[
            pl.BlockSpec((BLK_C, B * TIN), lambda i: (i, 0)),
            pl.BlockSpec((NC, BLK_C, B * TIN), lambda i: (0, i, 0)),
            pl.BlockSpec((2, 3 * DH), lambda i: (0, 0)),
            pl.BlockSpec((1, 3 * DH), lambda i: (0, 0)),
            pl.BlockSpec((DH, 3 * DH), lambda i: (0, 0)),
            pl.BlockSpec((DH, TOUT), lambda i: (0, 0)),
        ],
        out_specs=pl.BlockSpec((BLK_C, B * TOUT), lambda i: (i, 0)),
        out_shape=jax.ShapeDtypeStruct((N, B * TOUT), jnp.float32),
    )(x2, agg2, wx, bx2, wh, wdec)


def kernel(inputs, targets, entire_inputs, edge_index, W1, W2, Wf, bf, Wo,
           Wx, bx, Wh, Wdec):
    src = edge_index[0].astype(jnp.int32)
    dst = edge_index[1].astype(jnp.int32)

    u = jax.random.uniform(jax.random.key(42), (E, 2),
                           minval=1e-6, maxval=1.0 - 1e-6)
    g = -jnp.log(-jnp.log(u))
    gd = g[:, 0] - g[:, 1]

    wob = Wo[:, 0] - Wo[:, 1]
    x2 = jnp.transpose(inputs[:, :, :, 0], (2, 0, 1)).reshape(N, B * TIN)
    zero = jnp.zeros((RPT, B * TIN), jnp.float32)

    ha, hb = _encoder(entire_inputs, W1, W2, Wf[:128], Wf[128:],
                      bf.reshape(1, DH))
    mask, agg2 = _edge_sc(ha, hb, x2, src, dst, gd, wob, zero)
    out48 = _gru(x2, agg2, Wx, bx.reshape(1, 3 * DH), Wh, Wdec)
    outputs = out48.reshape(N, B, TOUT).transpose(1, 2, 0)[..., None]
    return (mask, outputs)


# speedup vs baseline: 48.7142x; 1.3461x over previous
import functools

import jax
import jax.numpy as jnp
import numpy as _np
from jax import lax
from jax.experimental import pallas as pl
from jax.experimental.pallas import tpu as pltpu
from jax.experimental.pallas import tpu_sc as plsc

N = 10000
E = 320000
B = 4
TIN = 12
TOUT = 12
DH = 64

NC = 2
NS = 16
NW = NC * NS
EW = E // NW
K = 128
NF = EW // K
KT = EW - NF * K
G = K // 16
DUMMY = N
AGGR = 10240
RPT = AGGR // NS

BLK_A = 256
BLK_C = 512


_UROWS = 2 * E // 128


def _enc_body(eint_ref, w1_ref, w2_ref, wfa_ref, wfb_ref, bf_ref,
              ha_ref, hb_ref):
    h1 = jnp.maximum(
        lax.dot_general(eint_ref[...], w1_ref[...],
                        (((0,), (0,)), ((), ())),
                        preferred_element_type=jnp.float32), 0.0)
    h = jnp.maximum(
        jnp.dot(h1, w2_ref[...], preferred_element_type=jnp.float32), 0.0)
    ha_ref[...] = (
        jnp.dot(h, wfa_ref[...], preferred_element_type=jnp.float32)
        + bf_ref[...])
    hb_ref[...] = jnp.dot(h, wfb_ref[...],
                          preferred_element_type=jnp.float32)


def _gum_body(u_ref, s_ref, gd_ref):
    g = -jnp.log(-jnp.log(u_ref[...]))
    gd_ref[...] = jnp.dot(g, s_ref[...], precision=lax.Precision.HIGHEST,
                          preferred_element_type=jnp.float32)


def _gumbel(u2d, sel):
    return pl.pallas_call(
        _gum_body,
        grid=(5,),
        in_specs=[
            pl.BlockSpec((_UROWS // 5, 128), lambda i: (i, 0)),
            pl.BlockSpec((128, 64), lambda i: (0, 0)),
        ],
        out_specs=pl.BlockSpec((_UROWS // 5, 64), lambda i: (i, 0)),
        out_shape=jax.ShapeDtypeStruct((_UROWS, 64), jnp.float32),
    )(u2d, sel)


def _encoder(eint, w1, w2, wfa, wfb, bf2):
    t = eint.shape[0]
    grid = pl.cdiv(N, BLK_A)
    return pl.pallas_call(
        _enc_body,
        grid=(grid,),
        in_specs=[
            pl.BlockSpec((t, BLK_A), lambda i: (0, i)),
            pl.BlockSpec((t, 256), lambda i: (0, 0)),
            pl.BlockSpec((256, 128), lambda i: (0, 0)),
            pl.BlockSpec((128, DH), lambda i: (0, 0)),
            pl.BlockSpec((128, DH), lambda i: (0, 0)),
            pl.BlockSpec((1, DH), lambda i: (0, 0)),
        ],
        out_specs=[
            pl.BlockSpec((BLK_A, DH), lambda i: (i, 0)),
            pl.BlockSpec((BLK_A, DH), lambda i: (i, 0)),
        ],
        out_shape=[
            jax.ShapeDtypeStruct((N, DH), jnp.float32),
            jax.ShapeDtypeStruct((N, DH), jnp.float32),
        ],
    )(eint, w1, w2, wfa, wfb, bf2)


def _edge_body(ha_hbm, hb_hbm, x_hbm, ei_hbm, gd_hbm, wob_hbm,
               zero_hbm, mask_hbm, agg_hbm,
               srcall, dstall, gdall, maskall, bufa, bufb, bufx, dstef,
               ta, tb, tx, tdst, wob, aggs, sem0, sem1):
    cid = lax.axis_index("c")
    sid = lax.axis_index("s")
    wid = cid * NS + sid
    ebase = wid * EW

    pltpu.sync_copy(zero_hbm, aggs.at[pl.ds(sid * RPT, RPT)])
    pltpu.sync_copy(wob_hbm, wob)
    pltpu.sync_copy(ei_hbm.at[pl.ds(ebase, EW)], srcall)
    pltpu.sync_copy(ei_hbm.at[pl.ds(E + ebase, EW)], dstall)
    pltpu.sync_copy(gd_hbm.at[pl.ds(ebase, EW)], gdall)
    plsc.subcore_barrier()

    sems = (sem0, sem1)

    def fire(j, p):
        sem = sems[p]
        soff = srcall.at[pl.ds(j * K, K)]
        doff = dstall.at[pl.ds(j * K, K)]
        pltpu.async_copy(ha_hbm.at[soff], bufa.at[p], sem)
        pltpu.async_copy(hb_hbm.at[doff], bufb.at[p], sem)
        pltpu.async_copy(x_hbm.at[soff], bufx.at[p], sem)

    def drain(j, p):
        sem = sems[p]
        soff = srcall.at[pl.ds(j * K, K)]
        doff = dstall.at[pl.ds(j * K, K)]
        pltpu.make_async_copy(ha_hbm.at[soff], bufa.at[p], sem).wait()
        pltpu.make_async_copy(hb_hbm.at[doff], bufb.at[p], sem).wait()
        pltpu.make_async_copy(x_hbm.at[soff], bufx.at[p], sem).wait()

    wc = [wob[pl.ds(16 * c, 16)] for c in range(DH // 16)]

    def score_group(ba, bb, ebos, g):
        base = ebos + g * 16
        gdv = gdall[pl.ds(base, 16)]
        dstv = dstall[pl.ds(base, 16)]
        lanes = lax.iota(jnp.int32, 16)
        tot = gdv
        for i in range(16):
            e = g * 16 + i
            t = []
            for c in range(DH // 16):
                va = ba[e, pl.ds(16 * c, 16)]
                vb = bb[e, pl.ds(16 * c, 16)]
                t.append(jnp.maximum(va + vb, 0.0) * wc[c])
            s = jnp.sum((t[0] + t[1]) + (t[2] + t[3]))
            tot = tot + jnp.where(lanes == i, s, 0.0)
        keep = tot >= 0.0
        maskall[pl.ds(base, 16)] = jnp.where(keep, 1.0, 0.0)
        return jnp.where(keep, dstv, jnp.full((16,), DUMMY, jnp.int32))

    def compute_store(j, p):
        ba = bufa.at[p]
        bb = bufb.at[p]

        def group(g, carry):
            dstef[p, pl.ds(g * 16, 16)] = score_group(ba, bb, j * K, g)
            return carry

        lax.fori_loop(0, G, group, 0)
        pltpu.sync_copy(bufx.at[p], aggs.at[dstef.at[p]], add=True)

    fire(0, 0)

    def dbl(i, carry):
        j0 = 2 * i
        fire(j0 + 1, 1)
        drain(j0, 0)
        compute_store(j0, 0)
        fire(j0 + 2, 0)
        drain(j0 + 1, 1)
        compute_store(j0 + 1, 1)
        return carry

    lax.fori_loop(0, NF // 2 - 1, dbl, 0)
    fire(NF - 1, 1)
    drain(NF - 2, 0)
    compute_store(NF - 2, 0)
    drain(NF - 1, 1)
    compute_store(NF - 1, 1)

    tsoff = srcall.at[pl.ds(NF * K, KT)]
    tdoff = dstall.at[pl.ds(NF * K, KT)]
    pltpu.async_copy(ha_hbm.at[tsoff], ta, sem0)
    pltpu.async_copy(hb_hbm.at[tdoff], tb, sem0)
    pltpu.async_copy(x_hbm.at[tsoff], tx, sem0)
    pltpu.make_async_copy(ha_hbm.at[tsoff], ta, sem0).wait()
    pltpu.make_async_copy(hb_hbm.at[tdoff], tb, sem0).wait()
    pltpu.make_async_copy(x_hbm.at[tsoff], tx, sem0).wait()
    tdst[...] = score_group(ta, tb, NF * K, 0)
    pltpu.sync_copy(tx, aggs.at[tdst], add=True)

    pltpu.sync_copy(maskall, mask_hbm.at[pl.ds(ebase, EW)])
    plsc.subcore_barrier()
    pltpu.sync_copy(aggs.at[pl.ds(sid * RPT, RPT)],
                    agg_hbm.at[cid, pl.ds(sid * RPT, RPT)])


def _edge_sc(ha, hb, x2, eiflat, gd, wob, zero):
    mesh = plsc.VectorSubcoreMesh(core_axis_name="c", subcore_axis_name="s")
    fn = pl.kernel(
        _edge_body,
        out_type=[
            jax.ShapeDtypeStruct((E,), jnp.float32),
            jax.ShapeDtypeStruct((NC, AGGR, B * TIN), jnp.float32),
        ],
        mesh=mesh,
        compiler_params=pltpu.CompilerParams(
            needs_layout_passes=False, use_tc_tiling_on_sc=False),
        scratch_types=[
            pltpu.VMEM((EW,), jnp.int32),
            pltpu.VMEM((EW,), jnp.int32),
            pltpu.VMEM((EW,), jnp.float32),
            pltpu.VMEM((EW,), jnp.float32),
            pltpu.VMEM((2, K, DH), jnp.float32),
            pltpu.VMEM((2, K, DH), jnp.float32),
            pltpu.VMEM((2, K, B * TIN), jnp.float32),
            pltpu.VMEM((2, K), jnp.int32),
            pltpu.VMEM((KT, DH), jnp.float32),
            pltpu.VMEM((KT, DH), jnp.float32),
            pltpu.VMEM((KT, B * TIN), jnp.float32),
            pltpu.VMEM((KT,), jnp.int32),
            pltpu.VMEM((DH,), jnp.float32),
            pltpu.VMEM_SHARED((AGGR, B * TIN), jnp.float32),
            pltpu.SemaphoreType.DMA,
            pltpu.SemaphoreType.DMA,
        ],
    )
    return fn(ha, hb, x2, eiflat, gd, wob, zero)


def _gru_body(x2_ref, agg_ref, wsel_ref, bx_ref, wh_ref, wdec_ref, out_ref):
    blk = x2_ref.shape[0]
    bx = bx_ref[...]
    wh = wh_ref[...]
    x2 = x2_ref[...]
    agg = agg_ref[0] + agg_ref[1]

    xa = jnp.concatenate(
        [jnp.concatenate([x2[:, b * TIN:(b + 1) * TIN] for b in range(B)],
                         axis=0),
         jnp.concatenate([agg[:, b * TIN:(b + 1) * TIN] for b in range(B)],
                         axis=0)], axis=1)
    h = jnp.zeros((B * blk, DH), jnp.float32)
    for t in range(TIN):
        xg = jnp.dot(xa, wsel_ref[t],
                     preferred_element_type=jnp.float32) + bx
        hg = jnp.dot(h, wh, preferred_element_type=jnp.float32)
        z = jax.nn.sigmoid(xg[:, :DH] + hg[:, :DH])
        r = jax.nn.sigmoid(xg[:, DH:2 * DH] + hg[:, DH:2 * DH])
        n = jnp.tanh(xg[:, 2 * DH:] + r * hg[:, 2 * DH:])
        h = n + z * (h - n)
    out = jnp.dot(h, wdec_ref[...], preferred_element_type=jnp.float32)
    for b in range(B):
        out_ref[:, b * TOUT:(b + 1) * TOUT] = out[b * blk:(b + 1) * blk, :]


def _gru(x2, agg2, wsel, bx2, wh, wdec):
    grid = pl.cdiv(N, BLK_C)
    return pl.pallas_call(
        _gru_body,
        grid=(grid,),
        in_specs=[
            pl.BlockSpec((BLK_C, B * TIN), lambda i: (i, 0)),
            pl.BlockSpec((NC, BLK_C, B * TIN), lambda i: (0, i, 0)),
            pl.BlockSpec((TIN, 2 * TIN, 3 * DH), lambda i: (0, 0, 0)),
            pl.BlockSpec((1, 3 * DH), lambda i: (0, 0)),
            pl.BlockSpec((DH, 3 * DH), lambda i: (0, 0)),
            pl.BlockSpec((DH, TOUT), lambda i: (0, 0)),
        ],
        out_specs=pl.BlockSpec((BLK_C, B * TOUT), lambda i: (i, 0)),
        out_shape=jax.ShapeDtypeStruct((N, B * TOUT), jnp.float32),
    )(x2, agg2, wsel, bx2, wh, wdec)


_SEL = _np.zeros((128, 64), _np.float32)
_SEL[2 * _np.arange(64), _np.arange(64)] = 1.0
_SEL[2 * _np.arange(64) + 1, _np.arange(64)] = -1.0


def kernel(inputs, targets, entire_inputs, edge_index, W1, W2, Wf, bf, Wo,
           Wx, bx, Wh, Wdec):
    eiflat = edge_index.astype(jnp.int32).reshape(2 * E)

    u = jax.random.uniform(jax.random.key(42), (E, 2),
                           minval=1e-6, maxval=1.0 - 1e-6)
    u2d = u.reshape(_UROWS, 128)

    wob = Wo[:, 0] - Wo[:, 1]
    x2 = jnp.transpose(inputs[:, :, :, 0], (2, 0, 1)).reshape(N, B * TIN)
    zero = jnp.zeros((RPT, B * TIN), jnp.float32)

    gd2 = _gumbel(u2d, jnp.asarray(_SEL))
    ha, hb = _encoder(entire_inputs.T, W1, W2, Wf[:128], Wf[128:],
                      bf.reshape(1, DH))
    mask, agg2 = _edge_sc(ha, hb, x2, eiflat, gd2.reshape(E), wob, zero)
    eye = jnp.eye(TIN, dtype=jnp.float32)
    wsel = (eye[:, :, None] * Wx[0][None, None, :])
    wsel = jnp.concatenate(
        [wsel, eye[:, :, None] * Wx[1][None, None, :]], axis=1)
    out48 = _gru(x2, agg2, wsel, bx.reshape(1, 3 * DH), Wh, Wdec)
    outputs = out48.reshape(N, B, TOUT).transpose(1, 2, 0)[..., None]
    return (mask, outputs)
